# baseline (speedup 1.0000x reference)
import jax
import jax.numpy as jnp
from jax.experimental import pallas as pl
from jax.experimental.pallas import tpu as pltpu
from jax.experimental.pallas import tpu_sc as plsc

_NUM_CHANNELS = 5
_GRID = 5


def _channel_mean(v):
    return (v * jnp.float32(_NUM_CHANNELS)) * jnp.float32(1.0 / _NUM_CHANNELS)


def _tc_body(e_ref, u_ref, eo_ref, uo_ref):
    ev = e_ref[...]
    cb = ev.shape[1]
    eo_ref[...] = ev.transpose(0, 3, 1, 2, 4).reshape(16, cb, 1, 128)
    uo_ref[...] = _channel_mean(u_ref[...])


def _edge_u_call(e4, u):
    de8, ncb, _, _, _ = e4.shape
    de = de8 * 8
    cb = ncb // _GRID
    d = u.shape[1]
    return pl.pallas_call(
        _tc_body,
        grid=(_GRID,),
        in_specs=[
            pl.BlockSpec((de8, cb, 1, 8, 128), lambda i: (0, i, 0, 0, 0)),
            pl.BlockSpec((1, d), lambda i: (0, 0)),
        ],
        out_specs=[
            pl.BlockSpec((de, cb, 1, 128), lambda i: (0, i, 0, 0)),
            pl.BlockSpec((1, d), lambda i: (0, 0)),
        ],
        out_shape=[
            jax.ShapeDtypeStruct((de, ncb, 1, 128), e4.dtype),
            jax.ShapeDtypeStruct((1, d), u.dtype),
        ],
        compiler_params=pltpu.CompilerParams(
            dimension_semantics=("arbitrary",),
        ),
    )(e4, u)


def _x_copy_sc(x):
    mesh = plsc.ScalarSubcoreMesh(axis_name="c")
    n, d = x.shape
    rows = n // mesh.num_cores

    @pl.kernel(
        out_type=jax.ShapeDtypeStruct(x.shape, x.dtype),
        mesh=mesh,
        scratch_types=[pltpu.SemaphoreType.DMA],
    )
    def sc_copy(x_hbm, o_hbm, sem):
        c = jax.lax.axis_index("c")
        r0 = c * rows
        pltpu.async_copy(
            x_hbm.at[pl.ds(r0, rows), :], o_hbm.at[pl.ds(r0, rows), :], sem
        ).wait()

    return sc_copy(x)


def kernel(x, edge_index, edge_attr, u, batch):
    n, d = x.shape
    e, de = edge_attr.shape

    et = edge_attr.T
    e4 = et.reshape(de // 8, 8, e // 128, 1, 128).transpose(0, 2, 3, 1, 4)

    e3, u_out = _edge_u_call(e4, u)
    x_out = _x_copy_sc(x)

    e_out = e3.reshape(de, e, 1).transpose(1, 0, 2)
    return (x_out[:, :, None], e_out, u_out[:, :, None])


# speedup vs baseline: 6.7269x; 6.7269x over previous
import jax
import jax.numpy as jnp
from jax.experimental import pallas as pl
from jax.experimental.pallas import tpu as pltpu

_NUM_CHANNELS = 5
_GRID = 5


def _scale():
    return jnp.float32(1.0 / _NUM_CHANNELS)


def _channel_mean(v):
    return (v * jnp.float32(_NUM_CHANNELS)) * _scale()


def _body(x_ref, e_ref, u_ref, xo_ref, eo_ref, uo_ref):
    ev = e_ref[...]
    cb = ev.shape[1]
    evt = ev.transpose(0, 3, 1, 2, 4).reshape(16, cb, 1, 128)
    eo_ref[...] = evt

    xo_ref[...] = _channel_mean(x_ref[...])
    uo_ref[...] = _channel_mean(u_ref[...])


def kernel(x, edge_index, edge_attr, u, batch):
    n, d = x.shape
    e, de = edge_attr.shape

    et = edge_attr.T
    e4 = et.reshape(de // 8, 8, e // 128, 1, 128).transpose(0, 2, 3, 1, 4)

    xb = n // _GRID
    cb = (e // 128) // _GRID

    x_out, e3, u_out = pl.pallas_call(
        _body,
        grid=(_GRID,),
        in_specs=[
            pl.BlockSpec((xb, d), lambda i: (i, 0)),
            pl.BlockSpec((de // 8, cb, 1, 8, 128), lambda i: (0, i, 0, 0, 0)),
            pl.BlockSpec((1, d), lambda i: (0, 0)),
        ],
        out_specs=[
            pl.BlockSpec((xb, d), lambda i: (i, 0)),
            pl.BlockSpec((de, cb, 1, 128), lambda i: (0, i, 0, 0)),
            pl.BlockSpec((1, d), lambda i: (0, 0)),
        ],
        out_shape=[
            jax.ShapeDtypeStruct((n, d), x.dtype),
            jax.ShapeDtypeStruct((de, e // 128, 1, 128), edge_attr.dtype),
            jax.ShapeDtypeStruct((1, d), u.dtype),
        ],
        compiler_params=pltpu.CompilerParams(
            dimension_semantics=("arbitrary",),
        ),
    )(x, e4, u)

    e_out = e3.reshape(de, e, 1).transpose(1, 0, 2)
    return (x_out[:, :, None], e_out, u_out[:, :, None])
